# baseline (speedup 1.0000x reference)
import dataclasses
import functools

import jax
import jax.numpy as jnp
from jax import lax
from jax.experimental import pallas as pl
from jax.experimental.pallas import tpu as pltpu
from jax.experimental.pallas import tpu_sc as plsc

_BATCH, _SEQ = 4, 8192
_NV = _SEQ // 16
_MAXM = 1229
_TVREGS = 77

_sc_params = pltpu.CompilerParams()
if "needs_layout_passes" in pltpu.CompilerParams.__dataclass_fields__:
    _sc_params = dataclasses.replace(_sc_params, needs_layout_passes=False)

_mesh = plsc.VectorSubcoreMesh(core_axis_name="c", subcore_axis_name="s")


def _ffs(mask):
    return jnp.max(plsc.all_reduce_ffs(mask))


def _scan512(hist_ref, r_s, iota):
    def body(i, carry):
        acc_s, found_s, bstar_s, G_s = carry
        vidx = 31 - i
        cv = hist_ref[pl.ds(vidx * 16, 16)]
        rv = lax.rev(cv, (0,))
        sfx = lax.cumsum(rv, axis=0)
        cross = (sfx + acc_s) >= r_s
        lane_s = _ffs(cross)
        hit = jnp.logical_and(found_s == 0, lane_s < 16)
        rv_l = jnp.sum(jnp.where(iota == lane_s, rv, 0))
        sfx_l = jnp.sum(jnp.where(iota == lane_s, sfx, 0))
        bstar_s = jnp.where(hit, vidx * 16 + 15 - lane_s, bstar_s)
        G_s = jnp.where(hit, acc_s + sfx_l - rv_l, G_s)
        found_s = jnp.where(hit, jnp.int32(1), found_s)
        acc_s = acc_s + jnp.sum(cv)
        return acc_s, found_s, bstar_s, G_s

    init = (jnp.int32(0), jnp.int32(0), jnp.int32(0), jnp.int32(0))
    _, _, bstar_s, G_s = lax.fori_loop(0, 32, body, init)
    return bstar_s, G_s


def _sc_body(inp_hbm, rand_hbm, out_hbm, inp_v, rand_v, u_v,
             h1_v, c1_v, h2_v, h3_v, res_v):
    wid = lax.axis_index("s") * 2 + lax.axis_index("c")

    @pl.when(wid < _BATCH)
    def _():
        pltpu.sync_copy(inp_hbm.at[wid], inp_v)
        pltpu.sync_copy(rand_hbm.at[wid], rand_v)
        zeros = jnp.zeros((16,), jnp.int32)
        ones = jnp.ones((16,), jnp.int32)
        iota = lax.iota(jnp.int32, 16)

        @pl.loop(0, _NV)
        def _(i):
            h1_v[pl.ds(i * 16, 16)] = zeros

        @pl.loop(0, 32)
        def _(i):
            c1_v[pl.ds(i * 16, 16)] = zeros
            h2_v[pl.ds(i * 16, 16)] = zeros
            h3_v[pl.ds(i * 16, 16)] = zeros

        def pass_a(i, acc):
            base = i * 64
            for k in range(4):
                off = base + k * 16
                t = inp_v[pl.ds(off, 16)]
                rn = rand_v[pl.ds(off, 16)]
                ign = (t == 0) | (t == 101) | (t == 102)
                bits = lax.bitcast_convert_type(rn, jnp.int32)
                u = jnp.where(ign, jnp.int32(0), bits + jnp.int32(1 << 30))
                u_v[pl.ds(off, 16)] = u
                acc = acc + jnp.where(ign, jnp.int32(0), jnp.int32(1))
                plsc.addupdate_scatter(h1_v, [lax.shift_right_logical(u, 18)], ones)
                plsc.addupdate_scatter(c1_v, [lax.shift_right_logical(u, 22)], ones)
            return acc

        acc = lax.fori_loop(0, _NV // 4, pass_a, zeros)
        num_s = jnp.sum(acc)
        tq = num_s.astype(jnp.float32) * jnp.float32(0.15)
        ti = tq.astype(jnp.int32)
        kq_s = jnp.where(ti.astype(jnp.float32) < tq, ti + 1, ti).astype(jnp.float32)

        def t_scan(j, carry):
            cum_s, p_s = carry
            uu = u_v[pl.ds(j * 16, 16)]
            cf = jnp.where(uu >= jnp.int32(1 << 30), jnp.int32(1), jnp.int32(0))
            incl = lax.cumsum(cf, axis=0)
            cross = ((cum_s + incl).astype(jnp.float32) > kq_s) & \
                    ((iota + j * 16) <= jnp.int32(_MAXM - 1))
            lane_s = _ffs(cross)
            hit = jnp.logical_and(p_s == jnp.int32(_SEQ), lane_s < 16)
            p_s = jnp.where(hit, j * 16 + lane_s, p_s)
            cum_s = cum_s + jnp.sum(cf)
            return cum_s, p_s

        _, p_s = lax.fori_loop(0, _TVREGS, t_scan,
                               (jnp.int32(0), jnp.int32(_SEQ)))
        t_b = jnp.minimum(p_s, jnp.int32(_MAXM))

        c1star_s, Gc_s = _scan512(c1_v, t_b, iota)
        cv = h1_v[pl.ds(c1star_s * 16, 16)]
        rv = lax.rev(cv, (0,))
        sfx = lax.cumsum(rv, axis=0)
        cross = (sfx + Gc_s) >= t_b
        lane_s = _ffs(cross)
        rv_l = jnp.sum(jnp.where(iota == lane_s, rv, 0))
        sfx_l = jnp.sum(jnp.where(iota == lane_s, sfx, 0))
        b1star_s = c1star_s * 16 + 15 - lane_s
        G1_s = Gc_s + sfx_l - rv_l
        r2_s = t_b - G1_s

        @pl.loop(0, _NV // 4)
        def _(i):
            base = i * 64
            for k in range(4):
                off = base + k * 16
                uu = u_v[pl.ds(off, 16)]
                m = lax.shift_right_logical(uu, 18) == b1star_s
                m2 = lax.shift_right_logical(uu, 9) & jnp.int32(511)
                plsc.addupdate_scatter(h2_v, [m2], ones, mask=m)

        m2star_s, G2_s = _scan512(h2_v, r2_s, iota)
        r3_s = r2_s - G2_s
        hi2_s = b1star_s * 512 + m2star_s

        @pl.loop(0, _NV // 4)
        def _(i):
            base = i * 64
            for k in range(4):
                off = base + k * 16
                uu = u_v[pl.ds(off, 16)]
                m = lax.shift_right_logical(uu, 9) == hi2_s
                m3 = uu & jnp.int32(511)
                plsc.addupdate_scatter(h3_v, [m3], ones, mask=m)

        m3star_s, G3_s = _scan512(h3_v, r3_s, iota)
        E_s = r3_s - G3_s
        vstar_s = hi2_s * 512 + m3star_s

        def pass_d(i, carry):
            acc_s, cstar_s = carry
            base = i * 64
            for k in range(4):
                off = base + k * 16
                uu = u_v[pl.ds(off, 16)]
                match = uu == vstar_s
                mi = jnp.where(match, jnp.int32(1), jnp.int32(0))
                incl = lax.cumsum(mi, axis=0)
                cross = match & ((acc_s + incl) >= E_s)
                lane_s = _ffs(cross)
                hit = jnp.logical_and(cstar_s == jnp.int32(_SEQ), lane_s < 16)
                cstar_s = jnp.where(hit, off + lane_s, cstar_s)
                acc_s = acc_s + jnp.sum(mi)
            return acc_s, cstar_s

        _, cstar_s = lax.fori_loop(0, _NV // 4, pass_d,
                                   (jnp.int32(0), jnp.int32(_SEQ)))

        res_v[...] = jnp.where(iota == 0, vstar_s, jnp.int32(0)) + \
                     jnp.where(iota == 1, cstar_s, jnp.int32(0))
        pltpu.sync_copy(res_v, out_hbm.at[wid])


_sc_select = functools.partial(
    pl.kernel,
    out_type=jax.ShapeDtypeStruct((_BATCH, 16), jnp.int32),
    mesh=_mesh,
    compiler_params=_sc_params,
    scratch_types=[
        pltpu.VMEM((_SEQ,), jnp.int32),
        pltpu.VMEM((_SEQ,), jnp.float32),
        pltpu.VMEM((_SEQ,), jnp.int32),
        pltpu.VMEM((_SEQ,), jnp.int32),
        pltpu.VMEM((512,), jnp.int32),
        pltpu.VMEM((512,), jnp.int32),
        pltpu.VMEM((512,), jnp.int32),
        pltpu.VMEM((16,), jnp.int32),
    ])(_sc_body)


def _tc_body(inp_ref, rand_ref, rep_ref, thr_ref, out_masked_ref, out_labels_ref):
    inp = inp_ref[...]
    rand = rand_ref[...]
    cand = jnp.logical_not((inp == 0) | (inp == 101) | (inp == 102))
    bits = lax.bitcast_convert_type(rand, jnp.int32)
    u = jnp.where(cand, bits + jnp.int32(1 << 30), jnp.int32(0))
    v_star = thr_ref[:, 0:1]
    c_star = thr_ref[:, 1:2]
    idx = lax.broadcasted_iota(jnp.int32, (_BATCH, _SEQ), 1)
    sel = (u > v_star) | ((u == v_star) & (idx <= c_star))
    rep = rep_ref[...] < jnp.float32(0.9)
    out_masked_ref[...] = jnp.where(sel & rep, jnp.int32(103), inp)
    out_labels_ref[...] = jnp.where(sel, inp, jnp.int32(0))


_tc_mask = pl.pallas_call(
    _tc_body,
    out_shape=(
        jax.ShapeDtypeStruct((_BATCH, _SEQ), jnp.int32),
        jax.ShapeDtypeStruct((_BATCH, _SEQ), jnp.int32),
    ),
)


@jax.jit
def kernel(input, rand_noise, replace_noise):
    thr = _sc_select(input, rand_noise)
    return _tc_mask(input, rand_noise, replace_noise, thr)
